# Initial kernel scaffold; baseline (speedup 1.0000x reference)
import jax
import jax.numpy as jnp
from jax.experimental import pallas as pl


def kernel(h, edge_index, snorm_n, params):
    raise NotImplementedError("write your pallas kernel here")


# speedup vs baseline: 4.3571x; 4.3571x over previous
import functools

import jax
import jax.numpy as jnp
from jax import lax
from jax.experimental import pallas as pl
from jax.experimental.pallas import tpu as pltpu
from jax.experimental.pallas import tpu_sc as plsc

_NC = 2
_NS = 16
_BN = 1000


def _segsum_sc(cur, src, dst):
    n, d = cur.shape
    e = src.shape[0]
    nw = _NC * _NS
    e_per_w = e // nw
    assert e_per_w * nw == e
    ch = 80
    n_chunks = e_per_w // ch
    assert n_chunks * ch == e_per_w
    rc = 400
    n_rchunks = n // rc
    assert n_rchunks * rc == n

    mesh = plsc.VectorSubcoreMesh(core_axis_name="c", subcore_axis_name="s")

    @functools.partial(
        pl.kernel,
        out_type=jax.ShapeDtypeStruct((_NC, n, d), jnp.float32),
        mesh=mesh,
        scratch_types=[
            pltpu.VMEM((ch,), jnp.int32),
            pltpu.VMEM((ch,), jnp.int32),
            pltpu.VMEM((ch, d), jnp.float32),
            pltpu.VMEM_SHARED((n, d), jnp.float32),
            pltpu.SemaphoreType.DMA,
        ],
    )
    def segsum(cur_hbm, src_hbm, dst_hbm, out_hbm,
               src_v, dst_v, rows_v, acc_sh, sem):
        c = lax.axis_index("c")
        s = lax.axis_index("s")
        wid = c * _NS + s

        def zrow(i, carry):
            def zcol(j, carry2):
                rows_v[i, pl.ds(j * 16, 16)] = jnp.zeros((16,), jnp.float32)
                return carry2
            return lax.fori_loop(0, d // 16, zcol, carry)
        lax.fori_loop(0, ch, zrow, 0)
        assert rc % ch == 0
        for k in range((n_rchunks + _NS - 1) // _NS):
            j = s + k * _NS

            def zchunk(jj):
                def zcopy(m, carry):
                    pltpu.sync_copy(
                        rows_v, acc_sh.at[pl.ds(jj * rc + m * ch, ch)])
                    return carry
                lax.fori_loop(0, rc // ch, zcopy, 0)
            if (k + 1) * _NS <= n_rchunks:
                zchunk(j)
            else:
                @pl.when(j < n_rchunks)
                def _():
                    zchunk(j)
        plsc.subcore_barrier()

        base = wid * e_per_w

        def chunk(i, carry):
            off = base + i * ch
            pltpu.sync_copy(src_hbm.at[pl.ds(off, ch)], src_v)
            pltpu.sync_copy(dst_hbm.at[pl.ds(off, ch)], dst_v)
            pltpu.async_copy(cur_hbm.at[src_v], rows_v, sem).wait()
            pltpu.sync_copy(rows_v, acc_sh.at[dst_v], add=True)
            return carry
        lax.fori_loop(0, n_chunks, chunk, 0)
        plsc.subcore_barrier()
        for k in range((n_rchunks + _NS - 1) // _NS):
            j = s + k * _NS
            if (k + 1) * _NS <= n_rchunks:
                pltpu.sync_copy(acc_sh.at[pl.ds(j * rc, rc)],
                                out_hbm.at[c, pl.ds(j * rc, rc)])
            else:
                @pl.when(j < n_rchunks)
                def _():
                    pltpu.sync_copy(acc_sh.at[pl.ds(j * rc, rc)],
                                    out_hbm.at[c, pl.ds(j * rc, rc)])

    return segsum(cur, src, dst)


def _dense_emb(h, w, b):
    n, d = h.shape
    nb = n // _BN
    assert nb * _BN == n

    def body(h_ref, w_ref, b_ref, out_ref, pooled_ref):
        x = jnp.dot(h_ref[...], w_ref[...],
                    preferred_element_type=jnp.float32) + b_ref[...]
        out_ref[...] = x

        @pl.when(pl.program_id(0) == 0)
        def _():
            pooled_ref[...] = jnp.zeros_like(pooled_ref)
        pooled_ref[...] += jnp.sum(x, axis=0, keepdims=True)

    return pl.pallas_call(
        body,
        grid=(nb,),
        in_specs=[
            pl.BlockSpec((_BN, d), lambda i: (i, 0)),
            pl.BlockSpec((d, d), lambda i: (0, 0)),
            pl.BlockSpec((1, d), lambda i: (0, 0)),
        ],
        out_specs=[
            pl.BlockSpec((_BN, d), lambda i: (i, 0)),
            pl.BlockSpec((1, d), lambda i: (0, 0)),
        ],
        out_shape=[
            jax.ShapeDtypeStruct((n, d), jnp.float32),
            jax.ShapeDtypeStruct((1, d), jnp.float32),
        ],
    )(h, w, b.reshape(1, d))


def _dense_mid(cur, part, w1, b1, w2, b2, snorm, eps):
    n, d = cur.shape
    nb = n // _BN

    def body(cur_ref, part_ref, w1_ref, b1_ref, w2_ref, b2_ref, sn_ref,
             eps_ref, u_ref, st_ref):
        z = (1.0 + eps_ref[...]) * cur_ref[...] + part_ref[0] + part_ref[1]
        a = jnp.maximum(
            jnp.dot(z, w1_ref[...], preferred_element_type=jnp.float32)
            + b1_ref[...], 0.0)
        u = (jnp.dot(a, w2_ref[...], preferred_element_type=jnp.float32)
             + b2_ref[...]) * sn_ref[...]
        u_ref[...] = u

        @pl.when(pl.program_id(0) == 0)
        def _():
            st_ref[...] = jnp.zeros_like(st_ref)
        st_ref[...] += jnp.concatenate(
            [jnp.sum(u, axis=0, keepdims=True),
             jnp.sum(u * u, axis=0, keepdims=True)], axis=0)

    return pl.pallas_call(
        body,
        grid=(nb,),
        in_specs=[
            pl.BlockSpec((_BN, d), lambda i: (i, 0)),
            pl.BlockSpec((_NC, _BN, d), lambda i: (0, i, 0)),
            pl.BlockSpec((d, d), lambda i: (0, 0)),
            pl.BlockSpec((1, d), lambda i: (0, 0)),
            pl.BlockSpec((d, d), lambda i: (0, 0)),
            pl.BlockSpec((1, d), lambda i: (0, 0)),
            pl.BlockSpec((_BN, 1), lambda i: (i, 0)),
            pl.BlockSpec((1, 1), lambda i: (0, 0)),
        ],
        out_specs=[
            pl.BlockSpec((_BN, d), lambda i: (i, 0)),
            pl.BlockSpec((2, d), lambda i: (0, 0)),
        ],
        out_shape=[
            jax.ShapeDtypeStruct((n, d), jnp.float32),
            jax.ShapeDtypeStruct((2, d), jnp.float32),
        ],
    )(cur, part, w1, b1.reshape(1, d), w2, b2.reshape(1, d), snorm,
      eps.reshape(1, 1))


def _dense_post(u, h_in, stats, gamma, beta):
    n, d = u.shape
    nb = n // _BN
    inv_n = 1.0 / n

    def body(u_ref, h_ref, st_ref, g_ref, bt_ref, out_ref, pooled_ref):
        mean = st_ref[0:1, :] * inv_n
        var = st_ref[1:2, :] * inv_n - mean * mean
        y = g_ref[...] * (u_ref[...] - mean) * lax.rsqrt(var + 1e-5) \
            + bt_ref[...]
        outv = h_ref[...] + jnp.maximum(y, 0.0)
        out_ref[...] = outv

        @pl.when(pl.program_id(0) == 0)
        def _():
            pooled_ref[...] = jnp.zeros_like(pooled_ref)
        pooled_ref[...] += jnp.sum(outv, axis=0, keepdims=True)

    return pl.pallas_call(
        body,
        grid=(nb,),
        in_specs=[
            pl.BlockSpec((_BN, d), lambda i: (i, 0)),
            pl.BlockSpec((_BN, d), lambda i: (i, 0)),
            pl.BlockSpec((2, d), lambda i: (0, 0)),
            pl.BlockSpec((1, d), lambda i: (0, 0)),
            pl.BlockSpec((1, d), lambda i: (0, 0)),
        ],
        out_specs=[
            pl.BlockSpec((_BN, d), lambda i: (i, 0)),
            pl.BlockSpec((1, d), lambda i: (0, 0)),
        ],
        out_shape=[
            jax.ShapeDtypeStruct((n, d), jnp.float32),
            jax.ShapeDtypeStruct((1, d), jnp.float32),
        ],
    )(u, h_in, stats, gamma.reshape(1, d), beta.reshape(1, d))


def _readout(pcat, wcat, bstack):
    c = wcat.shape[1]

    def body(p_ref, w_ref, b_ref, out_ref):
        out_ref[...] = jnp.dot(p_ref[...], w_ref[...],
                               preferred_element_type=jnp.float32) \
            + jnp.sum(b_ref[...], axis=0, keepdims=True)

    return pl.pallas_call(
        body,
        out_shape=jax.ShapeDtypeStruct((1, c), jnp.float32),
    )(pcat, wcat, bstack)


def kernel(h, edge_index, snorm_n, params):
    src = edge_index[0]
    dst = edge_index[1]
    cur, p0 = _dense_emb(h, params["W_emb"], params["b_emb"])
    pooled = [p0]
    for lp in params["layers"]:
        part = _segsum_sc(cur, src, dst)
        u, stats = _dense_mid(cur, part, lp["W1"], lp["b1"], lp["W2"],
                              lp["b2"], snorm_n, lp["eps"])
        cur, pi = _dense_post(u, cur, stats, lp["gamma"], lp["beta"])
        pooled.append(pi)
    pcat = jnp.concatenate(pooled, axis=1)
    wcat = jnp.concatenate(params["Wc"], axis=0)
    bstack = jnp.stack(params["bc"], axis=0)
    return _readout(pcat, wcat, bstack)


# speedup vs baseline: 9.7014x; 2.2266x over previous
import functools

import jax
import jax.numpy as jnp
from jax import lax
from jax.experimental import pallas as pl
from jax.experimental.pallas import tpu as pltpu
from jax.experimental.pallas import tpu_sc as plsc

_NC = 2
_NS = 16
_BN = 1000


def _segsum_sc(cur, src, dst):
    n, d = cur.shape
    e = src.shape[0]
    nw = _NC * _NS
    e_per_w = e // nw
    assert e_per_w * nw == e
    ch = 80
    n_chunks = e_per_w // ch
    assert n_chunks * ch == e_per_w
    rc = 400
    n_rchunks = n // rc
    assert n_rchunks * rc == n

    mesh = plsc.VectorSubcoreMesh(core_axis_name="c", subcore_axis_name="s")

    @functools.partial(
        pl.kernel,
        out_type=jax.ShapeDtypeStruct((_NC, n, d), jnp.float32),
        mesh=mesh,
        scratch_types=[
            pltpu.VMEM((e_per_w,), jnp.int32),
            pltpu.VMEM((n_chunks, ch), jnp.int32),
            pltpu.VMEM((ch, d), jnp.float32),
            pltpu.VMEM((ch, d), jnp.float32),
            pltpu.VMEM_SHARED((n, d), jnp.float32),
            pltpu.SemaphoreType.DMA,
            pltpu.SemaphoreType.DMA,
        ],
    )
    def segsum(cur_hbm, src_hbm, dst_hbm, out_hbm,
               src_v, dst_v, rows0_v, rows1_v, acc_sh, sem0, sem1):
        c = lax.axis_index("c")
        s = lax.axis_index("s")
        wid = c * _NS + s

        pltpu.sync_copy(src_hbm.at[pl.ds(wid * e_per_w, e_per_w)], src_v)
        pltpu.sync_copy(dst_hbm.at[wid], dst_v)

        def zrow(i, carry):
            def zcol(j, carry2):
                rows0_v[i, pl.ds(j * 16, 16)] = jnp.zeros((16,), jnp.float32)
                return carry2
            return lax.fori_loop(0, d // 16, zcol, carry)
        lax.fori_loop(0, ch, zrow, 0)
        assert rc % ch == 0
        for k in range((n_rchunks + _NS - 1) // _NS):
            j = s + k * _NS

            def zchunk(jj):
                def zcopy(m, carry):
                    pltpu.sync_copy(
                        rows0_v, acc_sh.at[pl.ds(jj * rc + m * ch, ch)])
                    return carry
                lax.fori_loop(0, rc // ch, zcopy, 0)
            if (k + 1) * _NS <= n_rchunks:
                zchunk(j)
            else:
                @pl.when(j < n_rchunks)
                def _():
                    zchunk(j)
        plsc.subcore_barrier()

        def sidx(i):
            return src_v.at[pl.ds(i * ch, ch)]

        pltpu.async_copy(cur_hbm.at[sidx(0)], rows0_v, sem0)

        def pipe(k, carry):
            i = 2 * k
            pltpu.async_copy(cur_hbm.at[sidx(i + 1)], rows1_v, sem1)
            pltpu.make_async_copy(cur_hbm.at[sidx(i)], rows0_v, sem0).wait()
            pltpu.sync_copy(rows0_v, acc_sh.at[dst_v.at[i]], add=True)

            @pl.when(i + 2 < n_chunks)
            def _():
                pltpu.async_copy(cur_hbm.at[sidx(i + 2)], rows0_v, sem0)
            pltpu.make_async_copy(cur_hbm.at[sidx(i + 1)],
                                  rows1_v, sem1).wait()
            pltpu.sync_copy(rows1_v, acc_sh.at[dst_v.at[i + 1]], add=True)
            return carry
        lax.fori_loop(0, n_chunks // 2, pipe, 0)
        if n_chunks % 2 == 1:
            last = n_chunks - 1
            pltpu.make_async_copy(cur_hbm.at[sidx(last)],
                                  rows0_v, sem0).wait()
            pltpu.sync_copy(rows0_v, acc_sh.at[dst_v.at[last]], add=True)
        plsc.subcore_barrier()
        for k in range((n_rchunks + _NS - 1) // _NS):
            j = s + k * _NS
            if (k + 1) * _NS <= n_rchunks:
                pltpu.sync_copy(acc_sh.at[pl.ds(j * rc, rc)],
                                out_hbm.at[c, pl.ds(j * rc, rc)])
            else:
                @pl.when(j < n_rchunks)
                def _():
                    pltpu.sync_copy(acc_sh.at[pl.ds(j * rc, rc)],
                                    out_hbm.at[c, pl.ds(j * rc, rc)])

    nw = _NC * _NS
    return segsum(cur, src, dst.reshape(nw, n_chunks, ch))


def _dense_emb(h, w, b):
    n, d = h.shape
    nb = n // _BN
    assert nb * _BN == n

    def body(h_ref, w_ref, b_ref, out_ref, pooled_ref):
        x = jnp.dot(h_ref[...], w_ref[...],
                    preferred_element_type=jnp.float32) + b_ref[...]
        out_ref[...] = x

        @pl.when(pl.program_id(0) == 0)
        def _():
            pooled_ref[...] = jnp.zeros_like(pooled_ref)
        pooled_ref[...] += jnp.sum(x, axis=0, keepdims=True)

    return pl.pallas_call(
        body,
        grid=(nb,),
        in_specs=[
            pl.BlockSpec((_BN, d), lambda i: (i, 0)),
            pl.BlockSpec((d, d), lambda i: (0, 0)),
            pl.BlockSpec((1, d), lambda i: (0, 0)),
        ],
        out_specs=[
            pl.BlockSpec((_BN, d), lambda i: (i, 0)),
            pl.BlockSpec((1, d), lambda i: (0, 0)),
        ],
        out_shape=[
            jax.ShapeDtypeStruct((n, d), jnp.float32),
            jax.ShapeDtypeStruct((1, d), jnp.float32),
        ],
    )(h, w, b.reshape(1, d))


def _dense_mid(cur, part, w1, b1, w2, b2, snorm, eps):
    n, d = cur.shape
    nb = n // _BN

    def body(cur_ref, part_ref, w1_ref, b1_ref, w2_ref, b2_ref, sn_ref,
             eps_ref, u_ref, st_ref):
        z = (1.0 + eps_ref[...]) * cur_ref[...] + part_ref[0] + part_ref[1]
        a = jnp.maximum(
            jnp.dot(z, w1_ref[...], preferred_element_type=jnp.float32)
            + b1_ref[...], 0.0)
        u = (jnp.dot(a, w2_ref[...], preferred_element_type=jnp.float32)
             + b2_ref[...]) * sn_ref[...]
        u_ref[...] = u

        @pl.when(pl.program_id(0) == 0)
        def _():
            st_ref[...] = jnp.zeros_like(st_ref)
        st_ref[...] += jnp.concatenate(
            [jnp.sum(u, axis=0, keepdims=True),
             jnp.sum(u * u, axis=0, keepdims=True)], axis=0)

    return pl.pallas_call(
        body,
        grid=(nb,),
        in_specs=[
            pl.BlockSpec((_BN, d), lambda i: (i, 0)),
            pl.BlockSpec((_NC, _BN, d), lambda i: (0, i, 0)),
            pl.BlockSpec((d, d), lambda i: (0, 0)),
            pl.BlockSpec((1, d), lambda i: (0, 0)),
            pl.BlockSpec((d, d), lambda i: (0, 0)),
            pl.BlockSpec((1, d), lambda i: (0, 0)),
            pl.BlockSpec((_BN, 1), lambda i: (i, 0)),
            pl.BlockSpec((1, 1), lambda i: (0, 0)),
        ],
        out_specs=[
            pl.BlockSpec((_BN, d), lambda i: (i, 0)),
            pl.BlockSpec((2, d), lambda i: (0, 0)),
        ],
        out_shape=[
            jax.ShapeDtypeStruct((n, d), jnp.float32),
            jax.ShapeDtypeStruct((2, d), jnp.float32),
        ],
    )(cur, part, w1, b1.reshape(1, d), w2, b2.reshape(1, d), snorm,
      eps.reshape(1, 1))


def _dense_post(u, h_in, stats, gamma, beta):
    n, d = u.shape
    nb = n // _BN
    inv_n = 1.0 / n

    def body(u_ref, h_ref, st_ref, g_ref, bt_ref, out_ref, pooled_ref):
        mean = st_ref[0:1, :] * inv_n
        var = st_ref[1:2, :] * inv_n - mean * mean
        y = g_ref[...] * (u_ref[...] - mean) * lax.rsqrt(var + 1e-5) \
            + bt_ref[...]
        outv = h_ref[...] + jnp.maximum(y, 0.0)
        out_ref[...] = outv

        @pl.when(pl.program_id(0) == 0)
        def _():
            pooled_ref[...] = jnp.zeros_like(pooled_ref)
        pooled_ref[...] += jnp.sum(outv, axis=0, keepdims=True)

    return pl.pallas_call(
        body,
        grid=(nb,),
        in_specs=[
            pl.BlockSpec((_BN, d), lambda i: (i, 0)),
            pl.BlockSpec((_BN, d), lambda i: (i, 0)),
            pl.BlockSpec((2, d), lambda i: (0, 0)),
            pl.BlockSpec((1, d), lambda i: (0, 0)),
            pl.BlockSpec((1, d), lambda i: (0, 0)),
        ],
        out_specs=[
            pl.BlockSpec((_BN, d), lambda i: (i, 0)),
            pl.BlockSpec((1, d), lambda i: (0, 0)),
        ],
        out_shape=[
            jax.ShapeDtypeStruct((n, d), jnp.float32),
            jax.ShapeDtypeStruct((1, d), jnp.float32),
        ],
    )(u, h_in, stats, gamma.reshape(1, d), beta.reshape(1, d))


def _readout(pcat, wcat, bstack):
    c = wcat.shape[1]

    def body(p_ref, w_ref, b_ref, out_ref):
        out_ref[...] = jnp.dot(p_ref[...], w_ref[...],
                               preferred_element_type=jnp.float32) \
            + jnp.sum(b_ref[...], axis=0, keepdims=True)

    return pl.pallas_call(
        body,
        out_shape=jax.ShapeDtypeStruct((1, c), jnp.float32),
    )(pcat, wcat, bstack)


def kernel(h, edge_index, snorm_n, params):
    src = edge_index[0]
    dst = edge_index[1]
    cur, p0 = _dense_emb(h, params["W_emb"], params["b_emb"])
    pooled = [p0]
    for lp in params["layers"]:
        part = _segsum_sc(cur, src, dst)
        u, stats = _dense_mid(cur, part, lp["W1"], lp["b1"], lp["W2"],
                              lp["b2"], snorm_n, lp["eps"])
        cur, pi = _dense_post(u, cur, stats, lp["gamma"], lp["beta"])
        pooled.append(pi)
    pcat = jnp.concatenate(pooled, axis=1)
    wcat = jnp.concatenate(params["Wc"], axis=0)
    bstack = jnp.stack(params["bc"], axis=0)
    return _readout(pcat, wcat, bstack)


# speedup vs baseline: 10.8747x; 1.1209x over previous
import functools

import jax
import jax.numpy as jnp
from jax import lax
from jax.experimental import pallas as pl
from jax.experimental.pallas import tpu as pltpu
from jax.experimental.pallas import tpu_sc as plsc

_NC = 2
_NS = 16
_BN = 1000


def _segsum_sc(cur, src, dst):
    n, d = cur.shape
    e = src.shape[0]
    nw = _NC * _NS
    e_per_w = e // nw
    assert e_per_w * nw == e
    ch = 80
    n_chunks = e_per_w // ch
    assert n_chunks * ch == e_per_w
    ph0 = ((n_chunks + 15) // 16) * 8
    st1 = ((n_chunks - ph0 + 7) // 8) * 8
    phases = [(0, ph0, ph0), (ph0, st1, n_chunks - ph0)]
    n_chunks_pad = ph0 + st1
    np_max = max(nst for _, nst, _ in phases)
    assert min(npc for _, _, npc in phases) >= 5
    rc = 400
    n_rchunks = n // rc
    assert n_rchunks * rc == n

    mesh = plsc.VectorSubcoreMesh(core_axis_name="c", subcore_axis_name="s")

    @functools.partial(
        pl.kernel,
        out_type=jax.ShapeDtypeStruct((_NC, n, d), jnp.float32),
        mesh=mesh,
        scratch_types=[
            pltpu.VMEM((np_max * ch,), jnp.int32),
            pltpu.VMEM((np_max, ch), jnp.int32),
            pltpu.VMEM((ch, d), jnp.float32),
            pltpu.VMEM((ch, d), jnp.float32),
            pltpu.VMEM((ch, d), jnp.float32),
            pltpu.VMEM_SHARED((n, d), jnp.float32),
            pltpu.SemaphoreType.DMA,
            pltpu.SemaphoreType.DMA,
            pltpu.SemaphoreType.DMA,
            pltpu.SemaphoreType.DMA,
            pltpu.SemaphoreType.DMA,
            pltpu.SemaphoreType.DMA,
        ],
    )
    def segsum(cur_hbm, src_hbm, dst_hbm, out_hbm,
               src_v, dst_v, r0, r1, r2, acc_sh,
               g0, g1, g2, s0, s1, s2):
        rows = [r0, r1, r2]
        gsem = [g0, g1, g2]
        ssem = [s0, s1, s2]
        rows0_v = r0
        c = lax.axis_index("c")
        s = lax.axis_index("s")
        wid = c * _NS + s

        def zrow(i, carry):
            def zcol(j, carry2):
                rows0_v[i, pl.ds(j * 16, 16)] = jnp.zeros((16,), jnp.float32)
                return carry2
            return lax.fori_loop(0, d // 16, zcol, carry)
        lax.fori_loop(0, ch, zrow, 0)
        assert rc % ch == 0
        for k in range((n_rchunks + _NS - 1) // _NS):
            j = s + k * _NS

            def zchunk(jj):
                def zcopy(m, carry):
                    pltpu.sync_copy(
                        rows0_v, acc_sh.at[pl.ds(jj * rc + m * ch, ch)])
                    return carry
                lax.fori_loop(0, rc // ch, zcopy, 0)
            if (k + 1) * _NS <= n_rchunks:
                zchunk(j)
            else:
                @pl.when(j < n_rchunks)
                def _():
                    zchunk(j)
        plsc.subcore_barrier()

        def sidx(i):
            return src_v.at[pl.ds(i * ch, ch)]

        def gwait(b):
            pltpu.make_async_copy(cur_hbm.at[sidx(0)], rows[b],
                                  gsem[b]).wait()

        def swait(b):
            pltpu.make_async_copy(rows[b], acc_sh.at[dst_v.at[0]],
                                  ssem[b]).wait()

        def gstart(i, b):
            pltpu.async_copy(cur_hbm.at[sidx(i)], rows[b], gsem[b])

        def sstart(i, b):
            pltpu.async_copy(rows[b], acc_sh.at[dst_v.at[i]],
                             ssem[b], add=True)

        def run_phase(off, nst, npc):
            pltpu.sync_copy(
                src_hbm.at[pl.ds(wid * e_per_w + off * ch, npc * ch)],
                src_v.at[pl.ds(0, npc * ch)])
            pltpu.sync_copy(dst_hbm.at[wid, pl.ds(off, nst)],
                            dst_v.at[pl.ds(0, nst)])

            gstart(0, 0)
            gstart(1, 1)
            gwait(0)
            sstart(0, 0)
            gstart(2, 2)

            n_main = ((npc - 1) // 3) * 3

            def pipe(g, carry):
                for b3 in range(3):
                    i = 1 + 3 * g + b3
                    bi = (1 + b3) % 3
                    bp = (bi + 2) % 3
                    gwait(bi)
                    sstart(i, bi)

                    @pl.when(i <= npc - 3)
                    def _():
                        swait(bp)
                        gstart(i + 2, bp)
                return carry
            lax.fori_loop(0, n_main // 3, pipe, 0)
            for i in range(1 + n_main, npc):
                gwait(i % 3)
                sstart(i, i % 3)
            for b in range(3):
                swait(b)

        for off, nst, npc in phases:
            run_phase(off, nst, npc)
        plsc.subcore_barrier()
        for k in range((n_rchunks + _NS - 1) // _NS):
            j = s + k * _NS
            if (k + 1) * _NS <= n_rchunks:
                pltpu.sync_copy(acc_sh.at[pl.ds(j * rc, rc)],
                                out_hbm.at[c, pl.ds(j * rc, rc)])
            else:
                @pl.when(j < n_rchunks)
                def _():
                    pltpu.sync_copy(acc_sh.at[pl.ds(j * rc, rc)],
                                    out_hbm.at[c, pl.ds(j * rc, rc)])

    dst_r = dst.reshape(nw, n_chunks, ch)
    if n_chunks_pad > n_chunks:
        dst_r = jnp.pad(dst_r, ((0, 0), (0, n_chunks_pad - n_chunks), (0, 0)))
    return segsum(cur, src, dst_r)


def _dense_emb(h, w, b):
    n, d = h.shape
    nb = n // _BN
    assert nb * _BN == n

    def body(h_ref, w_ref, b_ref, out_ref, pooled_ref):
        x = jnp.dot(h_ref[...], w_ref[...],
                    preferred_element_type=jnp.float32) + b_ref[...]
        out_ref[...] = x

        @pl.when(pl.program_id(0) == 0)
        def _():
            pooled_ref[...] = jnp.zeros_like(pooled_ref)
        pooled_ref[...] += jnp.sum(x, axis=0, keepdims=True)

    return pl.pallas_call(
        body,
        grid=(nb,),
        in_specs=[
            pl.BlockSpec((_BN, d), lambda i: (i, 0)),
            pl.BlockSpec((d, d), lambda i: (0, 0)),
            pl.BlockSpec((1, d), lambda i: (0, 0)),
        ],
        out_specs=[
            pl.BlockSpec((_BN, d), lambda i: (i, 0)),
            pl.BlockSpec((1, d), lambda i: (0, 0)),
        ],
        out_shape=[
            jax.ShapeDtypeStruct((n, d), jnp.float32),
            jax.ShapeDtypeStruct((1, d), jnp.float32),
        ],
    )(h, w, b.reshape(1, d))


def _dense_mid(cur, part, w1, b1, w2, b2, snorm, eps):
    n, d = cur.shape
    nb = n // _BN

    def body(cur_ref, part_ref, w1_ref, b1_ref, w2_ref, b2_ref, sn_ref,
             eps_ref, u_ref, st_ref):
        z = (1.0 + eps_ref[...]) * cur_ref[...] + part_ref[0] + part_ref[1]
        a = jnp.maximum(
            jnp.dot(z, w1_ref[...], preferred_element_type=jnp.float32)
            + b1_ref[...], 0.0)
        u = (jnp.dot(a, w2_ref[...], preferred_element_type=jnp.float32)
             + b2_ref[...]) * sn_ref[...]
        u_ref[...] = u

        @pl.when(pl.program_id(0) == 0)
        def _():
            st_ref[...] = jnp.zeros_like(st_ref)
        st_ref[...] += jnp.concatenate(
            [jnp.sum(u, axis=0, keepdims=True),
             jnp.sum(u * u, axis=0, keepdims=True)], axis=0)

    return pl.pallas_call(
        body,
        grid=(nb,),
        in_specs=[
            pl.BlockSpec((_BN, d), lambda i: (i, 0)),
            pl.BlockSpec((_NC, _BN, d), lambda i: (0, i, 0)),
            pl.BlockSpec((d, d), lambda i: (0, 0)),
            pl.BlockSpec((1, d), lambda i: (0, 0)),
            pl.BlockSpec((d, d), lambda i: (0, 0)),
            pl.BlockSpec((1, d), lambda i: (0, 0)),
            pl.BlockSpec((_BN, 1), lambda i: (i, 0)),
            pl.BlockSpec((1, 1), lambda i: (0, 0)),
        ],
        out_specs=[
            pl.BlockSpec((_BN, d), lambda i: (i, 0)),
            pl.BlockSpec((2, d), lambda i: (0, 0)),
        ],
        out_shape=[
            jax.ShapeDtypeStruct((n, d), jnp.float32),
            jax.ShapeDtypeStruct((2, d), jnp.float32),
        ],
    )(cur, part, w1, b1.reshape(1, d), w2, b2.reshape(1, d), snorm,
      eps.reshape(1, 1))


def _dense_post(u, h_in, stats, gamma, beta):
    n, d = u.shape
    nb = n // _BN
    inv_n = 1.0 / n

    def body(u_ref, h_ref, st_ref, g_ref, bt_ref, out_ref, pooled_ref):
        mean = st_ref[0:1, :] * inv_n
        var = st_ref[1:2, :] * inv_n - mean * mean
        y = g_ref[...] * (u_ref[...] - mean) * lax.rsqrt(var + 1e-5) \
            + bt_ref[...]
        outv = h_ref[...] + jnp.maximum(y, 0.0)
        out_ref[...] = outv

        @pl.when(pl.program_id(0) == 0)
        def _():
            pooled_ref[...] = jnp.zeros_like(pooled_ref)
        pooled_ref[...] += jnp.sum(outv, axis=0, keepdims=True)

    return pl.pallas_call(
        body,
        grid=(nb,),
        in_specs=[
            pl.BlockSpec((_BN, d), lambda i: (i, 0)),
            pl.BlockSpec((_BN, d), lambda i: (i, 0)),
            pl.BlockSpec((2, d), lambda i: (0, 0)),
            pl.BlockSpec((1, d), lambda i: (0, 0)),
            pl.BlockSpec((1, d), lambda i: (0, 0)),
        ],
        out_specs=[
            pl.BlockSpec((_BN, d), lambda i: (i, 0)),
            pl.BlockSpec((1, d), lambda i: (0, 0)),
        ],
        out_shape=[
            jax.ShapeDtypeStruct((n, d), jnp.float32),
            jax.ShapeDtypeStruct((1, d), jnp.float32),
        ],
    )(u, h_in, stats, gamma.reshape(1, d), beta.reshape(1, d))


def _readout(pcat, wcat, bstack):
    c = wcat.shape[1]

    def body(p_ref, w_ref, b_ref, out_ref):
        out_ref[...] = jnp.dot(p_ref[...], w_ref[...],
                               preferred_element_type=jnp.float32) \
            + jnp.sum(b_ref[...], axis=0, keepdims=True)

    return pl.pallas_call(
        body,
        out_shape=jax.ShapeDtypeStruct((1, c), jnp.float32),
    )(pcat, wcat, bstack)


def kernel(h, edge_index, snorm_n, params):
    src = edge_index[0]
    dst = edge_index[1]
    cur, p0 = _dense_emb(h, params["W_emb"], params["b_emb"])
    pooled = [p0]
    for lp in params["layers"]:
        part = _segsum_sc(cur, src, dst)
        u, stats = _dense_mid(cur, part, lp["W1"], lp["b1"], lp["W2"],
                              lp["b2"], snorm_n, lp["eps"])
        cur, pi = _dense_post(u, cur, stats, lp["gamma"], lp["beta"])
        pooled.append(pi)
    pcat = jnp.concatenate(pooled, axis=1)
    wcat = jnp.concatenate(params["Wc"], axis=0)
    bstack = jnp.stack(params["bc"], axis=0)
    return _readout(pcat, wcat, bstack)
